# baseline (speedup 1.0000x reference)
import functools

import jax
import jax.numpy as jnp
from jax import lax
from jax.experimental import pallas as pl
from jax.experimental.pallas import tpu as pltpu
from jax.experimental.pallas import tpu_sc as plsc

NC = 2
NS = 16
NW = NC * NS
L = 16


def _sc_tile_pos_embed(xf, embf, ar2, gate16, n_tiles, n_tok, E, MM):
    CH = E // L
    TPW = n_tok // NW
    TAIL = n_tok - NW * TPW

    mesh = plsc.VectorSubcoreMesh(
        core_axis_name="c", subcore_axis_name="s",
        num_cores=NC, num_subcores=NS)

    @functools.partial(
        pl.kernel,
        out_type=jax.ShapeDtypeStruct((n_tok * E,), jnp.float32),
        mesh=mesh,
        scratch_types=[
            pltpu.VMEM((2, L), jnp.int32),
            pltpu.VMEM((L,), jnp.float32),
            pltpu.VMEM((E,), jnp.float32),
            pltpu.VMEM((TPW * E,), jnp.float32),
            pltpu.VMEM((max(TAIL, 1) * E,), jnp.float32),
            pltpu.SemaphoreType.DMA,
            pltpu.SemaphoreType.DMA,
        ],
    )
    def k(x_hbm, emb_hbm, ar_hbm, g_hbm, out_hbm, arv, gv, ebuf, buf, tbuf,
          sem, sem2):
        wid = lax.axis_index("s") * NC + lax.axis_index("c")

        pltpu.sync_copy(ar_hbm, arv)
        pltpu.sync_copy(g_hbm, gv)
        vidx = arv[0, :] * arv[1, :] - 1
        idx_s = vidx[0]
        tile = jnp.clip(idx_s, 0, n_tiles - 1)
        erow = jnp.clip(idx_s, 0, MM - 1)

        xcopy = pltpu.async_copy(
            x_hbm.at[pl.ds((tile * n_tok + wid * TPW) * E, TPW * E)],
            buf, sem)
        ecopy = pltpu.async_copy(emb_hbm.at[pl.ds(erow * E, E)], ebuf, sem2)

        sx = jnp.full((L,), jnp.where(idx_s < n_tiles, 1.0, 0.0),
                      dtype=jnp.float32)
        g = gv[...]
        tg = 1.0 - 2.0 / (jnp.exp(g * 2.0) + 1.0)

        ecopy.wait()
        for ci in range(CH):
            ebuf[pl.ds(ci * L, L)] = ebuf[pl.ds(ci * L, L)] * tg

        xcopy.wait()

        def col_body(ci, carry):
            off = ci * L
            eg = ebuf[pl.ds(off, L)]
            for r in range(TPW):
                buf[pl.ds(r * E + off, L)] = buf[pl.ds(r * E + off, L)] * sx + eg
            return carry

        lax.fori_loop(0, CH, col_body, 0)
        pltpu.sync_copy(buf, out_hbm.at[pl.ds(wid * TPW * E, TPW * E)])

        if TAIL:
            @pl.when(wid == 0)
            def _():
                pltpu.async_copy(
                    x_hbm.at[pl.ds((tile * n_tok + NW * TPW) * E, TAIL * E)],
                    tbuf, sem).wait()

                def tcol_body(ci, carry):
                    off = ci * L
                    eg = ebuf[pl.ds(off, L)]
                    for r in range(TAIL):
                        tbuf[pl.ds(r * E + off, L)] = (
                            tbuf[pl.ds(r * E + off, L)] * sx + eg)
                    return carry

                lax.fori_loop(0, CH, tcol_body, 0)
                pltpu.sync_copy(
                    tbuf, out_hbm.at[pl.ds(NW * TPW * E, TAIL * E)])

    return k(xf, embf, ar2, gate16)


def kernel(x, aspect_ratio, embedding, gate):
    bsz, n_tiles, n_tok, E = x.shape
    M = embedding.shape[0]
    xf = x.reshape(bsz * n_tiles * n_tok * E)
    embf = embedding.astype(jnp.float32).reshape(M * M * E)
    ar2 = jnp.broadcast_to(
        aspect_ratio.astype(jnp.int32).reshape(2, 1), (2, L))
    gate16 = jnp.broadcast_to(gate.astype(jnp.float32).reshape(1), (L,))
    out = _sc_tile_pos_embed(xf, embf, ar2, gate16, n_tiles, n_tok, E, M * M)
    return out.reshape(bsz, 1, n_tok, E)


# speedup vs baseline: 2.2786x; 2.2786x over previous
import functools

import jax
import jax.numpy as jnp
from jax import lax
from jax.experimental import pallas as pl
from jax.experimental.pallas import tpu as pltpu
from jax.experimental.pallas import tpu_sc as plsc

NC = 2
NS = 16
NW = NC * NS
L = 16


def _sc_tile_pos_embed(x3, emb4, ar2, gate16, n_tiles, n_tok, E, M):
    CH = E // L
    TPW = n_tok // NW
    TAIL = n_tok - NW * TPW

    mesh = plsc.VectorSubcoreMesh(
        core_axis_name="c", subcore_axis_name="s",
        num_cores=NC, num_subcores=NS)

    @functools.partial(
        pl.kernel,
        out_type=jax.ShapeDtypeStruct((n_tok, E), jnp.float32),
        mesh=mesh,
        scratch_types=[
            pltpu.VMEM((2, L), jnp.int32),
            pltpu.VMEM((L,), jnp.float32),
            pltpu.VMEM((E,), jnp.float32),
            pltpu.VMEM((TPW, E), jnp.float32),
            pltpu.VMEM((max(TAIL, 1), E), jnp.float32),
            pltpu.SemaphoreType.DMA,
            pltpu.SemaphoreType.DMA,
        ],
    )
    def k(x_hbm, emb_hbm, ar_hbm, g_hbm, out_hbm, arv, gv, ebuf, buf, tbuf,
          sem, sem2):
        wid = lax.axis_index("s") * NC + lax.axis_index("c")

        pltpu.sync_copy(ar_hbm, arv)
        pltpu.sync_copy(g_hbm, gv)
        vidx = arv[0, :] * arv[1, :] - 1
        idx_s = vidx[0]
        tile = jnp.clip(idx_s, 0, n_tiles - 1)
        erow = jnp.clip(idx_s, 0, M * M - 1)

        xcopy = pltpu.async_copy(
            x_hbm.at[tile, pl.ds(wid * TPW, TPW), :], buf, sem)
        ecopy = pltpu.async_copy(
            emb_hbm.at[erow // M, erow % M, 0, :], ebuf, sem2)

        sx = jnp.full((L,), jnp.where(idx_s < n_tiles, 1.0, 0.0),
                      dtype=jnp.float32)
        g = gv[...]
        tg = 1.0 - 2.0 / (jnp.exp(g * 2.0) + 1.0)

        ecopy.wait()
        for ci in range(CH):
            ebuf[pl.ds(ci * L, L)] = ebuf[pl.ds(ci * L, L)] * tg

        xcopy.wait()

        def col_body(ci, carry):
            off = ci * L
            eg = ebuf[pl.ds(off, L)]
            for r in range(TPW):
                buf[r, pl.ds(off, L)] = buf[r, pl.ds(off, L)] * sx + eg
            return carry

        lax.fori_loop(0, CH, col_body, 0)
        pltpu.sync_copy(buf, out_hbm.at[pl.ds(wid * TPW, TPW), :])

        if TAIL:
            @pl.when(wid == 0)
            def _():
                pltpu.async_copy(
                    x_hbm.at[tile, pl.ds(NW * TPW, TAIL), :], tbuf, sem).wait()

                def tcol_body(ci, carry):
                    off = ci * L
                    eg = ebuf[pl.ds(off, L)]
                    for r in range(TAIL):
                        tbuf[r, pl.ds(off, L)] = tbuf[r, pl.ds(off, L)] * sx + eg
                    return carry

                lax.fori_loop(0, CH, tcol_body, 0)
                pltpu.sync_copy(tbuf, out_hbm.at[pl.ds(NW * TPW, TAIL), :])

    return k(x3, emb4, ar2, gate16)


def kernel(x, aspect_ratio, embedding, gate):
    bsz, n_tiles, n_tok, E = x.shape
    M = embedding.shape[0]
    x3 = x.reshape(n_tiles, n_tok, E)
    emb4 = embedding.astype(jnp.float32)
    ar2 = jnp.broadcast_to(
        aspect_ratio.astype(jnp.int32).reshape(2, 1), (2, L))
    gate16 = jnp.broadcast_to(gate.astype(jnp.float32).reshape(1), (L,))
    out = _sc_tile_pos_embed(x3, emb4, ar2, gate16, n_tiles, n_tok, E, M)
    return out.reshape(bsz, 1, n_tok, E)


# speedup vs baseline: 7.5466x; 3.3119x over previous
import functools

import jax
import jax.numpy as jnp
from jax import lax
from jax.experimental import pallas as pl
from jax.experimental.pallas import tpu as pltpu
from jax.experimental.pallas import tpu_sc as plsc

NC = 2
NS = 16
NW = NC * NS
L = 16


def _sc_tile_pos_embed(xt, embr, ar2, gate16, n_tiles, n_tok, E, MM):
    TPW = n_tok // NW
    TAIL = n_tok - NW * TPW
    NCH = 2
    CHT = TPW // NCH
    CL = E // L
    assert TPW % NCH == 0

    mesh = plsc.VectorSubcoreMesh(
        core_axis_name="c", subcore_axis_name="s",
        num_cores=NC, num_subcores=NS)

    @functools.partial(
        pl.kernel,
        out_type=jax.ShapeDtypeStruct((n_tok, 1, E), jnp.float32),
        mesh=mesh,
        scratch_types=[
            pltpu.VMEM((2, L), jnp.int32),
            pltpu.VMEM((L,), jnp.float32),
            pltpu.VMEM((1, 1, E), jnp.float32),
            pltpu.VMEM((CHT, n_tiles, E), jnp.float32),
            pltpu.VMEM((CHT, 1, E), jnp.float32),
            pltpu.SemaphoreType.DMA,
            pltpu.SemaphoreType.DMA,
        ],
    )
    def k(x_hbm, emb_hbm, ar_hbm, g_hbm, out_hbm, arv, gv, ebuf, buf, cbuf,
          sem, sem2):
        wid = lax.axis_index("s") * NC + lax.axis_index("c")
        tok0 = wid * TPW

        pltpu.sync_copy(ar_hbm, arv)
        pltpu.sync_copy(g_hbm, gv)
        vidx = arv[0, :] * arv[1, :] - 1
        idx_s = vidx[0]
        tile = jnp.clip(idx_s, 0, n_tiles - 1)
        erow = jnp.clip(idx_s, 0, MM - 1)

        xcopy = pltpu.async_copy(
            x_hbm.at[0, pl.ds(tok0, CHT), :, :], buf, sem)
        ecopy = pltpu.async_copy(emb_hbm.at[pl.ds(erow, 1), :, :], ebuf, sem2)

        sx = jnp.full((L,), jnp.where(idx_s < n_tiles, 1.0, 0.0),
                      dtype=jnp.float32)
        g = gv[...]
        tg = 1.0 - 2.0 / (jnp.exp(g * 2.0) + 1.0)

        ecopy.wait()
        for ci in range(CL):
            o = ci * L
            ebuf[0, 0, pl.ds(o, L)] = ebuf[0, 0, pl.ds(o, L)] * tg

        def compact(nt):
            def tok_body(t, carry):
                for ci in range(CL):
                    o = ci * L
                    cbuf[t, 0, pl.ds(o, L)] = (
                        buf[t, tile, pl.ds(o, L)] * sx
                        + ebuf[0, 0, pl.ds(o, L)])
                return carry
            lax.fori_loop(0, nt, tok_body, 0)

        for ch in range(NCH):
            xcopy.wait()
            compact(CHT)
            if ch + 1 < NCH:
                nxt = tok0 + (ch + 1) * CHT
                xcopy = pltpu.async_copy(
                    x_hbm.at[0, pl.ds(nxt, CHT), :, :], buf, sem)
            pltpu.sync_copy(
                cbuf, out_hbm.at[pl.ds(tok0 + ch * CHT, CHT), :, :])

        if TAIL:
            @pl.when(wid == 0)
            def _():
                pltpu.async_copy(
                    x_hbm.at[0, pl.ds(NW * TPW, TAIL), :, :],
                    buf.at[pl.ds(0, TAIL), :, :], sem).wait()
                compact(TAIL)
                pltpu.sync_copy(
                    cbuf.at[pl.ds(0, TAIL), :, :],
                    out_hbm.at[pl.ds(NW * TPW, TAIL), :, :])

    return k(xt, embr, ar2, gate16)


def kernel(x, aspect_ratio, embedding, gate):
    bsz, n_tiles, n_tok, E = x.shape
    M = embedding.shape[0]
    xt = x.transpose(0, 2, 1, 3)
    embr = embedding.astype(jnp.float32).reshape(M * M, 1, E)
    ar2 = jnp.broadcast_to(
        aspect_ratio.astype(jnp.int32).reshape(2, 1), (2, L))
    gate16 = jnp.broadcast_to(gate.astype(jnp.float32).reshape(1), (L,))
    out = _sc_tile_pos_embed(xt, embr, ar2, gate16, n_tiles, n_tok, E, M * M)
    return out.reshape(bsz, 1, n_tok, E)


# speedup vs baseline: 8.6850x; 1.1509x over previous
import functools

import jax
import jax.numpy as jnp
from jax import lax
from jax.experimental import pallas as pl
from jax.experimental.pallas import tpu as pltpu
from jax.experimental.pallas import tpu_sc as plsc

NC = 2
NS = 16
NW = NC * NS
L = 16
W = 128


def _sc_tile_pos_embed(xt, embr, ar2, gate16, n_tiles, n_tok, E, MM):
    TPW = n_tok // NW
    TAIL = n_tok - NW * TPW
    NCH = 4
    CHT = TPW // NCH
    RPT = E // W
    assert TPW % NCH == 0 and TAIL <= CHT

    mesh = plsc.VectorSubcoreMesh(
        core_axis_name="c", subcore_axis_name="s",
        num_cores=NC, num_subcores=NS)

    @functools.partial(
        pl.kernel,
        out_type=jax.ShapeDtypeStruct((n_tok, 1, E), jnp.float32),
        mesh=mesh,
        scratch_types=[
            pltpu.VMEM((2, L), jnp.int32),
            pltpu.VMEM((L,), jnp.float32),
            pltpu.VMEM((1, 1, E), jnp.float32),
            pltpu.VMEM((CHT, n_tiles, E), jnp.float32),
            pltpu.VMEM((CHT, n_tiles, E), jnp.float32),
            pltpu.VMEM((CHT, 1, E), jnp.float32),
            pltpu.VMEM((CHT, 1, E), jnp.float32),
            pltpu.SemaphoreType.DMA,
            pltpu.SemaphoreType.DMA,
            pltpu.SemaphoreType.DMA,
        ],
    )
    def k(x_hbm, emb_hbm, ar_hbm, g_hbm, out_hbm, arv, gv, ebuf, buf0, buf1,
          cbuf0, cbuf1, semx, seme, semo):
        wid = lax.axis_index("s") * NC + lax.axis_index("c")
        tok0 = wid * TPW
        bufs = (buf0, buf1)
        cbufs = (cbuf0, cbuf1)

        pltpu.sync_copy(ar_hbm, arv)
        pltpu.sync_copy(g_hbm, gv)
        vidx = arv[0, :] * arv[1, :] - 1
        idx_s = vidx[0]
        tile = jnp.clip(idx_s, 0, n_tiles - 1)
        erow = jnp.clip(idx_s, 0, MM - 1)

        incopies = [None] * NCH
        for ch in range(min(2, NCH)):
            incopies[ch] = pltpu.async_copy(
                x_hbm.at[0, pl.ds(tok0 + ch * CHT, CHT), :, :],
                bufs[ch % 2], semx)
        ecopy = pltpu.async_copy(emb_hbm.at[pl.ds(erow, 1), :, :], ebuf, seme)

        sx = jnp.full((L,), jnp.where(idx_s < n_tiles, 1.0, 0.0),
                      dtype=jnp.float32)
        g = gv[...]
        tg = 1.0 - 2.0 / (jnp.exp(g * 2.0) + 1.0)

        ecopy.wait()
        for ci in range(E // L):
            o = ci * L
            ebuf[0, 0, pl.ds(o, L)] = ebuf[0, 0, pl.ds(o, L)] * tg

        def compact(buf, cbuf, nt):
            def eb_body(eb, carry):
                for ci in range(W // L):
                    o = eb * W + ci * L
                    eg = ebuf[0, 0, pl.ds(o, L)]
                    for t in range(nt):
                        cbuf[t, 0, pl.ds(o, L)] = (
                            buf[t, tile, pl.ds(o, L)] * sx + eg)
                return carry
            lax.fori_loop(0, RPT, eb_body, 0)

        outcopies = [None] * NCH
        for ch in range(NCH):
            incopies[ch].wait()
            if ch >= 2:
                outcopies[ch - 2].wait()
            compact(bufs[ch % 2], cbufs[ch % 2], CHT)
            if ch + 2 < NCH:
                incopies[ch + 2] = pltpu.async_copy(
                    x_hbm.at[0, pl.ds(tok0 + (ch + 2) * CHT, CHT), :, :],
                    bufs[ch % 2], semx)
            outcopies[ch] = pltpu.async_copy(
                cbufs[ch % 2],
                out_hbm.at[pl.ds(tok0 + ch * CHT, CHT), :, :], semo)

        if TAIL:
            @pl.when(wid == 0)
            def _():
                tcopy = pltpu.async_copy(
                    x_hbm.at[0, pl.ds(NW * TPW, TAIL), :, :],
                    buf0.at[pl.ds(0, TAIL), :, :], semx)
                tcopy.wait()
                outcopies[NCH - 2].wait()
                compact(buf0, cbuf0, TAIL)
                pltpu.sync_copy(
                    cbuf0.at[pl.ds(0, TAIL), :, :],
                    out_hbm.at[pl.ds(NW * TPW, TAIL), :, :])

        if TAIL:
            @pl.when(wid != 0)
            def _():
                outcopies[NCH - 2].wait()
            outcopies[NCH - 1].wait()
        else:
            outcopies[NCH - 2].wait()
            outcopies[NCH - 1].wait()

    return k(xt, embr, ar2, gate16)


def kernel(x, aspect_ratio, embedding, gate):
    bsz, n_tiles, n_tok, E = x.shape
    M = embedding.shape[0]
    xt = x.transpose(0, 2, 1, 3)
    embr = embedding.astype(jnp.float32).reshape(M * M, 1, E)
    ar2 = jnp.broadcast_to(
        aspect_ratio.astype(jnp.int32).reshape(2, 1), (2, L))
    gate16 = jnp.broadcast_to(gate.astype(jnp.float32).reshape(1), (L,))
    out = _sc_tile_pos_embed(xt, embr, ar2, gate16, n_tiles, n_tok, E, M * M)
    return out.reshape(bsz, 1, n_tok, E)
